# baseline (speedup 1.0000x reference)
import jax
import jax.numpy as jnp
from jax import lax
from jax.experimental import pallas as pl
from jax.experimental.pallas import tpu as pltpu
from jax.experimental.pallas import tpu_sc as plsc

NUM_INPUTS = 100000
D = 128
B = 16384
K = 50

NC = 2
NS = 16
NW = NC * NS
RPW = B // NW
BLK = 128
NBLK = RPW // BLK
SUB = 32
NSUB = BLK // SUB
LANES = 16
DCH = D // LANES

_BCAST_DNUMS = lax.GatherDimensionNumbers(
    offset_dims=(), collapsed_slice_dims=(0,), start_index_map=(0,))


def _lane_broadcast(vec, lane):
    idx = jnp.full((LANES, 1), lane, dtype=jnp.int32)
    return lax.gather(vec, idx, _BCAST_DNUMS, (1,),
                      mode=lax.GatherScatterMode.PROMISE_IN_BOUNDS)


def _sc_body(idx_hbm, vals_hbm, weight_hbm, bias_hbm, out_hbm,
             idx_v, vals_v, rows_v, acc_v, bias_v, gsem, osem):
    wid = lax.axis_index("s") * NC + lax.axis_index("c")
    col0 = wid * RPW

    pltpu.sync_copy(bias_hbm, bias_v)
    bias_c = tuple(bias_v[pl.ds(j * LANES, LANES)] for j in range(DCH))

    @pl.loop(0, NBLK)
    def _blk_loop(blk):
        base = col0 + blk * BLK

        def fire_gather(kk, sub, buf):
            pltpu.async_copy(
                weight_hbm.at[idx_v.at[kk, pl.ds(sub * SUB, SUB)]],
                rows_v.at[buf], gsem.at[buf])

        def wait_gather(kk, sub, buf):
            pltpu.make_async_copy(
                weight_hbm.at[idx_v.at[kk, pl.ds(sub * SUB, SUB)]],
                rows_v.at[buf], gsem.at[buf]).wait()

        pltpu.sync_copy(idx_hbm.at[:, pl.ds(base, BLK)], idx_v)
        pltpu.sync_copy(vals_hbm.at[:, pl.ds(base, BLK)], vals_v)

        fire_gather(0, 0, 0)

        @pl.when(blk >= 1)
        def _():
            pltpu.make_async_copy(acc_v, out_hbm.at[pl.ds(col0, BLK)],
                                  osem).wait()

        @pl.loop(0, BLK)
        def _init(rr):
            for j in range(DCH):
                acc_v[rr, pl.ds(j * LANES, LANES)] = bias_c[j]

        @pl.loop(0, K)
        def _k_loop(kk):
            for sub in range(NSUB):
                if sub + 1 < NSUB:
                    fire_gather(kk, sub + 1, (sub + 1) % 2)
                else:
                    @pl.when(kk + 1 < K)
                    def _():
                        fire_gather(kk + 1, 0, (sub + 1) % 2)

                wait_gather(kk, sub, sub % 2)

                for bc in range(SUB // LANES):
                    vb16 = vals_v[kk, pl.ds(sub * SUB + bc * LANES, LANES)]
                    for i in range(LANES):
                        vb = _lane_broadcast(vb16, i)
                        arow = sub * SUB + bc * LANES + i
                        grow = bc * LANES + i
                        for j in range(DCH):
                            plsc.addupdate(
                                acc_v.at[arow, pl.ds(j * LANES, LANES)],
                                rows_v[sub % 2, grow,
                                       pl.ds(j * LANES, LANES)] * vb)

        pltpu.async_copy(acc_v, out_hbm.at[pl.ds(base, BLK)], osem)

    pltpu.make_async_copy(acc_v, out_hbm.at[pl.ds(col0, BLK)], osem).wait()


@jax.jit
def kernel(feature_indices, feature_values, weight, bias):
    idx_t = feature_indices.T
    vals_t = feature_values.T

    mesh = plsc.VectorSubcoreMesh(core_axis_name="c", subcore_axis_name="s")
    run = pl.kernel(
        _sc_body,
        out_type=jax.ShapeDtypeStruct((B, D), jnp.float32),
        mesh=mesh,
        scratch_types=[
            pltpu.VMEM((K, BLK), jnp.int32),
            pltpu.VMEM((K, BLK), jnp.float32),
            pltpu.VMEM((2, SUB, D), jnp.float32),
            pltpu.VMEM((BLK, D), jnp.float32),
            pltpu.VMEM((D,), jnp.float32),
            pltpu.SemaphoreType.DMA((2,)),
            pltpu.SemaphoreType.DMA,
        ],
    )
    return run(idx_t, vals_t, weight, bias)


# speedup vs baseline: 2.0857x; 2.0857x over previous
import jax
import jax.numpy as jnp
from jax import lax
from jax.experimental import pallas as pl
from jax.experimental.pallas import tpu as pltpu
from jax.experimental.pallas import tpu_sc as plsc

NUM_INPUTS = 100000
D = 128
B = 16384
K = 50

NC = 2
NS = 16
NW = NC * NS
RPW = B // NW
BLK = 128
NBLK = RPW // BLK
SUB = 64
NSUB = BLK // SUB
LANES = 16
DCH = D // LANES

_BCAST_DNUMS = lax.GatherDimensionNumbers(
    offset_dims=(), collapsed_slice_dims=(0,), start_index_map=(0,))


def _lane_broadcast(vec, lane):
    idx = jnp.full((LANES, 1), lane, dtype=jnp.int32)
    return lax.gather(vec, idx, _BCAST_DNUMS, (1,),
                      mode=lax.GatherScatterMode.PROMISE_IN_BOUNDS)


def _sc_body(idx_hbm, vals_hbm, weight_hbm, bias_hbm, out_hbm,
             idx_v, vals_v, rows_v, acc_v, bias_v, gsem, osem):
    wid = lax.axis_index("s") * NC + lax.axis_index("c")
    col0 = wid * RPW

    pltpu.sync_copy(bias_hbm, bias_v)
    bias_c = tuple(bias_v[pl.ds(j * LANES, LANES)] for j in range(DCH))

    @pl.loop(0, NBLK)
    def _blk_loop(blk):
        base = col0 + blk * BLK

        def fire_gather(kk, sub, buf):
            pltpu.async_copy(
                weight_hbm.at[idx_v.at[kk, pl.ds(sub * SUB, SUB)]],
                rows_v.at[buf], gsem.at[buf])

        def wait_gather(kk, sub, buf):
            pltpu.make_async_copy(
                weight_hbm.at[idx_v.at[kk, pl.ds(sub * SUB, SUB)]],
                rows_v.at[buf], gsem.at[buf]).wait()

        pltpu.sync_copy(idx_hbm.at[:, pl.ds(base, BLK)], idx_v)
        pltpu.sync_copy(vals_hbm.at[:, pl.ds(base, BLK)], vals_v)

        fire_gather(0, 0, 0)

        @pl.when(blk >= 1)
        def _():
            pltpu.make_async_copy(acc_v, out_hbm.at[pl.ds(col0, BLK)],
                                  osem).wait()

        @pl.loop(0, BLK)
        def _init(rr):
            for j in range(DCH):
                acc_v[rr, pl.ds(j * LANES, LANES)] = bias_c[j]

        @pl.loop(0, K)
        def _k_loop(kk):
            for sub in range(NSUB):
                if sub + 1 < NSUB:
                    fire_gather(kk, sub + 1, (sub + 1) % 2)
                else:
                    @pl.when(kk + 1 < K)
                    def _():
                        fire_gather(kk + 1, 0, (sub + 1) % 2)

                wait_gather(kk, sub, sub % 2)

                for bc in range(SUB // LANES):
                    vb16 = vals_v[kk, pl.ds(sub * SUB + bc * LANES, LANES)]
                    for i in range(LANES):
                        vb = _lane_broadcast(vb16, i)
                        arow = sub * SUB + bc * LANES + i
                        grow = bc * LANES + i
                        for j in range(DCH):
                            ds = pl.ds(j * LANES, LANES)
                            acc_v[arow, ds] = (
                                acc_v[arow, ds]
                                + rows_v[sub % 2, grow, ds] * vb)

        pltpu.async_copy(acc_v, out_hbm.at[pl.ds(base, BLK)], osem)

    pltpu.make_async_copy(acc_v, out_hbm.at[pl.ds(col0, BLK)], osem).wait()


@jax.jit
def kernel(feature_indices, feature_values, weight, bias):
    idx_t = feature_indices.T
    vals_t = feature_values.T

    mesh = plsc.VectorSubcoreMesh(core_axis_name="c", subcore_axis_name="s")
    run = pl.kernel(
        _sc_body,
        out_type=jax.ShapeDtypeStruct((B, D), jnp.float32),
        mesh=mesh,
        scratch_types=[
            pltpu.VMEM((K, BLK), jnp.int32),
            pltpu.VMEM((K, BLK), jnp.float32),
            pltpu.VMEM((2, SUB, D), jnp.float32),
            pltpu.VMEM((BLK, D), jnp.float32),
            pltpu.VMEM((D,), jnp.float32),
            pltpu.SemaphoreType.DMA((2,)),
            pltpu.SemaphoreType.DMA,
        ],
    )
    return run(idx_t, vals_t, weight, bias)


# speedup vs baseline: 6.3028x; 3.0219x over previous
import functools

import jax
import jax.numpy as jnp
from jax import lax
from jax.experimental import pallas as pl
from jax.experimental.pallas import tpu as pltpu
from jax.experimental.pallas import tpu_sc as plsc

NUM_INPUTS = 100000
D = 128
B = 16384
K = 50
KPAD = 64

NC = 2
NS = 16
NW = NC * NS
RPW = B // NW
GRP = 2
NG = RPW // GRP
LANES = 16
DCH = D // LANES

_BCAST_DNUMS = lax.GatherDimensionNumbers(
    offset_dims=(), collapsed_slice_dims=(0,), start_index_map=(0,))


def _lane_broadcast(vec, lane):
    idx = jnp.full((LANES, 1), lane, dtype=jnp.int32)
    return lax.gather(vec, idx, _BCAST_DNUMS, (1,),
                      mode=lax.GatherScatterMode.PROMISE_IN_BOUNDS)


def _sc_body(idx_hbm, vals_hbm, weight_hbm, bias_hbm, out_hbm,
             idx_v, vals_v, rows_v, bias_v, out_v, gsem, osem):
    wid = lax.axis_index("s") * NC + lax.axis_index("c")
    row0 = wid * RPW
    grp0 = wid * NG

    pltpu.sync_copy(idx_hbm.at[pl.ds(grp0, NG)], idx_v)
    pltpu.sync_copy(vals_hbm.at[pl.ds(row0 * KPAD, RPW * KPAD)], vals_v)
    pltpu.sync_copy(bias_hbm, bias_v)

    def fire_gather(grp, buf):
        pltpu.async_copy(weight_hbm.at[idx_v.at[grp]], rows_v.at[buf],
                         gsem.at[buf])

    def wait_gather(grp, buf):
        pltpu.make_async_copy(weight_hbm.at[idx_v.at[grp]], rows_v.at[buf],
                              gsem.at[buf]).wait()

    def out_slice(grp):
        return out_hbm.at[pl.ds(row0 + grp * GRP, GRP)]

    for pg in range(3):
        fire_gather(pg, pg)

    @pl.loop(0, NG, step=4)
    def _grp_loop(g):
        for b in range(4):
            grp = g + b

            @pl.when(grp + 3 < NG)
            def _():
                fire_gather(grp + 3, (b + 3) % 4)

            wait_gather(grp, b)

            if b >= 2:
                pltpu.make_async_copy(out_v.at[b % 2], out_slice(grp),
                                      osem.at[b % 2]).wait()
            else:
                @pl.when(g >= 4)
                def _():
                    pltpu.make_async_copy(out_v.at[b % 2], out_slice(grp),
                                          osem.at[b % 2]).wait()

            for r in range(GRP):
                rloc = grp * GRP + r
                accs = tuple(bias_v[pl.ds(j * LANES, LANES)]
                             for j in range(DCH))
                for t in range(KPAD // LANES):
                    kcnt = min(LANES, K - t * LANES)
                    if kcnt <= 0:
                        break
                    voff = pl.multiple_of(rloc * KPAD + t * LANES, LANES)
                    vv_t = vals_v[pl.ds(voff, LANES)]

                    @pl.loop(0, kcnt, init_carry=accs, unroll=4)
                    def _k_loop(lane, accs, r=r, b=b, t=t, vv_t=vv_t):
                        vb = _lane_broadcast(vv_t, lane)
                        krow = r * K + t * LANES + lane
                        return tuple(
                            accs[j] + rows_v[b, krow,
                                             pl.ds(j * LANES, LANES)] * vb
                            for j in range(DCH))

                    accs = _k_loop
                for j in range(DCH):
                    out_v[b % 2, r, pl.ds(j * LANES, LANES)] = accs[j]

            pltpu.async_copy(out_v.at[b % 2], out_slice(grp), osem.at[b % 2])

    for b in range(2):
        pltpu.make_async_copy(out_v.at[b], out_hbm.at[pl.ds(row0, GRP)],
                              osem.at[b]).wait()


@jax.jit
def kernel(feature_indices, feature_values, weight, bias):
    idx2 = feature_indices.reshape(B // GRP, GRP * K)
    vals_p = jnp.pad(feature_values, ((0, 0), (0, KPAD - K))).reshape(B * KPAD)

    mesh = plsc.VectorSubcoreMesh(core_axis_name="c", subcore_axis_name="s")
    run = pl.kernel(
        _sc_body,
        out_type=jax.ShapeDtypeStruct((B, D), jnp.float32),
        mesh=mesh,
        scratch_types=[
            pltpu.VMEM((NG, GRP * K), jnp.int32),
            pltpu.VMEM((RPW * KPAD,), jnp.float32),
            pltpu.VMEM((4, GRP * K, D), jnp.float32),
            pltpu.VMEM((D,), jnp.float32),
            pltpu.VMEM((2, GRP, D), jnp.float32),
            pltpu.SemaphoreType.DMA((4,)),
            pltpu.SemaphoreType.DMA((2,)),
        ],
    )
    return run(idx2, vals_p, weight, bias)


# speedup vs baseline: 6.3871x; 1.0134x over previous
import functools

import jax
import jax.numpy as jnp
from jax import lax
from jax.experimental import pallas as pl
from jax.experimental.pallas import tpu as pltpu
from jax.experimental.pallas import tpu_sc as plsc

NUM_INPUTS = 100000
D = 128
B = 16384
K = 50
KPAD = 64

NC = 2
NS = 16
NW = NC * NS
RPW = B // NW
GRP = 2
NG = RPW // GRP
LANES = 16
DCH = D // LANES

_BCAST_DNUMS = lax.GatherDimensionNumbers(
    offset_dims=(), collapsed_slice_dims=(0,), start_index_map=(0,))


def _lane_broadcast(vec, lane):
    idx = jnp.full((LANES, 1), lane, dtype=jnp.int32)
    return lax.gather(vec, idx, _BCAST_DNUMS, (1,),
                      mode=lax.GatherScatterMode.PROMISE_IN_BOUNDS)


def _sc_body(idx_hbm, vals_hbm, weight_hbm, bias_hbm, out_hbm,
             idx_v, vals_v, rows_v, bias_v, out_v, gsem, osem):
    wid = lax.axis_index("s") * NC + lax.axis_index("c")
    row0 = wid * RPW
    grp0 = wid * NG

    pltpu.sync_copy(idx_hbm.at[pl.ds(grp0, NG)], idx_v)
    pltpu.sync_copy(vals_hbm.at[pl.ds(row0 * K, RPW * K)], vals_v)
    pltpu.sync_copy(bias_hbm, bias_v)

    def fire_gather(grp, buf):
        pltpu.async_copy(weight_hbm.at[idx_v.at[grp]], rows_v.at[buf],
                         gsem.at[buf])

    def wait_gather(grp, buf):
        pltpu.make_async_copy(weight_hbm.at[idx_v.at[grp]], rows_v.at[buf],
                              gsem.at[buf]).wait()

    def out_slice(grp):
        return out_hbm.at[pl.ds(row0 + grp * GRP, GRP)]

    for pg in range(3):
        fire_gather(pg, pg)

    @pl.loop(0, NG, step=4)
    def _grp_loop(g):
        for b in range(4):
            grp = g + b

            @pl.when(grp + 3 < NG)
            def _():
                fire_gather(grp + 3, (b + 3) % 4)

            wait_gather(grp, b)

            if b >= 2:
                pltpu.make_async_copy(out_v.at[b % 2], out_slice(grp),
                                      osem.at[b % 2]).wait()
            else:
                @pl.when(g >= 4)
                def _():
                    pltpu.make_async_copy(out_v.at[b % 2], out_slice(grp),
                                          osem.at[b % 2]).wait()

            for r in range(GRP):
                rloc = grp * GRP + r
                accs = tuple(bias_v[pl.ds(j * LANES, LANES)]
                             for j in range(DCH))
                for woff, kbase, kcnt in ((0, 0, 16), (16, 16, 16),
                                          (32, 32, 16), (34, 48, 2)):
                    voff = pl.multiple_of(rloc * K + woff, 2)
                    vv_t = vals_v[pl.ds(voff, LANES)]

                    @pl.loop(0, kcnt, init_carry=accs, unroll=4)
                    def _k_loop(lane, accs, r=r, b=b, vv_t=vv_t,
                                woff=woff, kbase=kbase):
                        vb = _lane_broadcast(vv_t, kbase - woff + lane)
                        krow = r * K + kbase + lane
                        return tuple(
                            accs[j] + rows_v[b, krow,
                                             pl.ds(j * LANES, LANES)] * vb
                            for j in range(DCH))

                    accs = _k_loop
                for j in range(DCH):
                    out_v[b % 2, r, pl.ds(j * LANES, LANES)] = accs[j]

            pltpu.async_copy(out_v.at[b % 2], out_slice(grp), osem.at[b % 2])

    for b in range(2):
        pltpu.make_async_copy(out_v.at[b], out_hbm.at[pl.ds(row0, GRP)],
                              osem.at[b]).wait()


@jax.jit
def kernel(feature_indices, feature_values, weight, bias):
    idx2 = feature_indices.reshape(B // GRP, GRP * K)
    vals_p = feature_values.reshape(B * K)

    mesh = plsc.VectorSubcoreMesh(core_axis_name="c", subcore_axis_name="s")
    run = pl.kernel(
        _sc_body,
        out_type=jax.ShapeDtypeStruct((B, D), jnp.float32),
        mesh=mesh,
        scratch_types=[
            pltpu.VMEM((NG, GRP * K), jnp.int32),
            pltpu.VMEM((RPW * K,), jnp.float32),
            pltpu.VMEM((4, GRP * K, D), jnp.float32),
            pltpu.VMEM((D,), jnp.float32),
            pltpu.VMEM((2, GRP, D), jnp.float32),
            pltpu.SemaphoreType.DMA((4,)),
            pltpu.SemaphoreType.DMA((2,)),
        ],
    )
    return run(idx2, vals_p, weight, bias)
